# Initial kernel scaffold; baseline (speedup 1.0000x reference)
import jax
import jax.numpy as jnp
from jax.experimental import pallas as pl


def kernel(x, edges_idx, batch_idx, g_features, Wl0, bl0, Wr0, Wl1, bl1, Wr1, Wg, bg, Wo, bo):
    raise NotImplementedError("write your pallas kernel here")


# speedup vs baseline: 6.7273x; 6.7273x over previous
import functools

import jax
import jax.numpy as jnp
from jax import lax
from jax.experimental import pallas as pl
from jax.experimental.pallas import tpu as pltpu
from jax.experimental.pallas import tpu_sc as plsc

NC = 2
NS = 16
EDGE_CHUNK = 400
LANES = 16


def _pad_rows(n):
  return ((n + NS * 8 - 1) // (NS * 8)) * NS * 8


def _static_chunks(total, chunk):
  out, off = [], 0
  while off < total:
    sz = min(chunk, total - off)
    out.append((off, sz))
    off += sz
  return out


def _sc_agg_body(with_cnt, n, e, hd, *refs):
  if with_cnt:
    (feat, src, dst, agg_out, cnt_out,
     sidx_v, didx_v, rows_v, ones_v, acc_sh, cnt_sh, sem) = refs
  else:
    (feat, src, dst, agg_out,
     sidx_v, didx_v, rows_v, acc_sh, sem) = refs
  c = lax.axis_index("c")
  s = lax.axis_index("s")
  n_pad = _pad_rows(n)
  rows_per_tile = n_pad // NS
  ch = EDGE_CHUNK

  def zrow(i, _):
    for j in range(hd // LANES):
      rows_v[i, pl.ds(j * LANES, LANES)] = jnp.zeros((LANES,), jnp.float32)
    return 0
  lax.fori_loop(0, ch, zrow, 0)

  base = s * rows_per_tile
  for off, sz in _static_chunks(rows_per_tile, ch):
    pltpu.sync_copy(rows_v.at[pl.ds(0, sz)], acc_sh.at[pl.ds(base + off, sz)])

  if with_cnt:
    def zone(i, _):
      ones_v[i, :] = jnp.zeros((LANES,), jnp.float32)
      return 0
    lax.fori_loop(0, ch, zone, 0)

    @pl.when(c == 0)
    def _zero_cnt():
      for off, sz in _static_chunks(rows_per_tile, ch):
        pltpu.sync_copy(ones_v.at[pl.ds(0, sz)],
                        cnt_sh.at[pl.ds(base + off, sz)])

    def sone(i, _):
      ones_v[i, :] = jnp.ones((LANES,), jnp.float32)
      return 0
    lax.fori_loop(0, ch, sone, 0)

  plsc.subcore_barrier()

  e_per_tile = e // NS
  n_chunks = e_per_tile // ch
  ebase = s * e_per_tile
  row_off = c * n

  def body(j, _):
    off = ebase + j * ch
    pltpu.sync_copy(src.at[pl.ds(off, ch)], sidx_v)
    pltpu.sync_copy(dst.at[pl.ds(off, ch)], didx_v)
    for k in range(ch // LANES):
      sl = pl.ds(k * LANES, LANES)
      sidx_v[sl] = sidx_v[sl] + row_off
    pltpu.async_copy(feat.at[sidx_v], rows_v, sem).wait()
    pltpu.sync_copy(rows_v, acc_sh.at[didx_v], add=True)
    if with_cnt:
      @pl.when(c == 0)
      def _cnt_add():
        pltpu.sync_copy(ones_v, cnt_sh.at[didx_v], add=True)
    return 0
  lax.fori_loop(0, n_chunks, body, 0)

  plsc.subcore_barrier()

  pltpu.sync_copy(acc_sh.at[pl.ds(base, rows_per_tile)],
                  agg_out.at[c, pl.ds(base, rows_per_tile)])
  if with_cnt:
    @pl.when(c == 0)
    def _cnt_out():
      pltpu.sync_copy(cnt_sh.at[pl.ds(base, rows_per_tile)],
                      cnt_out.at[0, pl.ds(base, rows_per_tile)])


def _make_sc_agg(with_cnt, n, e, hd):
  mesh = plsc.VectorSubcoreMesh(core_axis_name="c", subcore_axis_name="s")
  n_pad = _pad_rows(n)
  out_type = [jax.ShapeDtypeStruct((NC, n_pad, hd), jnp.float32)]
  scratch = [
      pltpu.VMEM((EDGE_CHUNK,), jnp.int32),
      pltpu.VMEM((EDGE_CHUNK,), jnp.int32),
      pltpu.VMEM((EDGE_CHUNK, hd), jnp.float32),
  ]
  if with_cnt:
    out_type.append(jax.ShapeDtypeStruct((1, n_pad, LANES), jnp.float32))
    scratch.append(pltpu.VMEM((EDGE_CHUNK, LANES), jnp.float32))
  scratch.append(pltpu.VMEM_SHARED((n_pad, hd), jnp.float32))
  if with_cnt:
    scratch.append(pltpu.VMEM_SHARED((n_pad, LANES), jnp.float32))
  scratch.append(pltpu.SemaphoreType.DMA)
  return pl.kernel(
      functools.partial(_sc_agg_body, with_cnt, n, e, hd),
      out_type=tuple(out_type),
      mesh=mesh,
      scratch_types=tuple(scratch),
      compiler_params=pltpu.CompilerParams(use_tc_tiling_on_sc=False),
  )


def _split_halves(h, n, hd):
  return jnp.stack([h[:, :hd], h[:, hd:]])


def _tc_layer_body(relu, agg_ref, cntp_ref, x_ref, wlT_ref, wrT_ref, bl_ref,
                   o_ref):
  agg = jnp.concatenate([agg_ref[0], agg_ref[1]], axis=1)
  x = jnp.concatenate([x_ref[0], x_ref[1]], axis=1)
  cnt = cntp_ref[0, :, 0]
  rc = 1.0 / jnp.maximum(cnt, 1.0)
  mean = agg * rc[:, None]
  h = (jnp.dot(mean, wlT_ref[...], preferred_element_type=jnp.float32)
       + jnp.dot(x, wrT_ref[...], preferred_element_type=jnp.float32)
       + bl_ref[...])
  h = jnp.maximum(h, 0.0) if relu else h
  hd = h.shape[1] // 2
  o_ref[0] = h[:, :hd]
  o_ref[1] = h[:, hd:]


def _make_tc_layer(relu, n, d, block):
  grid = n // block
  n_pad = _pad_rows(n)
  hd = d // 2
  return pl.pallas_call(
      functools.partial(_tc_layer_body, relu),
      grid=(grid,),
      in_specs=[
          pl.BlockSpec((NC, block, hd), lambda i: (0, i, 0)),
          pl.BlockSpec((1, block, LANES), lambda i: (0, i, 0)),
          pl.BlockSpec((NC, block, hd), lambda i: (0, i, 0)),
          pl.BlockSpec((d, d), lambda i: (0, 0)),
          pl.BlockSpec((d, d), lambda i: (0, 0)),
          pl.BlockSpec((1, d), lambda i: (0, 0)),
      ],
      out_specs=pl.BlockSpec((NC, block, hd), lambda i: (0, i, 0)),
      out_shape=jax.ShapeDtypeStruct((NC, n, hd), jnp.float32),
  )


def _tc_final_body(g, block, agg_ref, cntp_ref, x_ref, wlT_ref, wrT_ref,
                   bl_ref, batch_ref, gf_ref, wgT_ref, bg_ref, woaT_ref,
                   wobT_ref, wocT_ref, bo_ref, o_ref, sum_acc, cnt_acc,
                   max_acc):
  i = pl.program_id(0)
  nblocks = pl.num_programs(0)

  @pl.when(i == 0)
  def _init():
    sum_acc[...] = jnp.zeros_like(sum_acc)
    cnt_acc[...] = jnp.zeros_like(cnt_acc)
    max_acc[...] = jnp.full_like(max_acc, -jnp.inf)

  agg = jnp.concatenate([agg_ref[0], agg_ref[1]], axis=1)
  x = jnp.concatenate([x_ref[0], x_ref[1]], axis=1)
  cnt = cntp_ref[0, :, 0]
  rc = 1.0 / jnp.maximum(cnt, 1.0)
  mean = agg * rc[:, None]
  h = (jnp.dot(mean, wlT_ref[...], preferred_element_type=jnp.float32)
       + jnp.dot(x, wrT_ref[...], preferred_element_type=jnp.float32)
       + bl_ref[...])

  bcol = batch_ref[...]
  gids = lax.broadcasted_iota(jnp.int32, (block, g), 1)
  onehot = (bcol == gids).astype(jnp.float32)
  sum_acc[...] += lax.dot_general(
      onehot, h, (((0,), (0,)), ((), ())),
      preferred_element_type=jnp.float32)
  cnt_acc[...] += lax.dot_general(
      onehot, jnp.ones_like(h), (((0,), (0,)), ((), ())),
      preferred_element_type=jnp.float32)

  for gg in range(g):
    mg = jnp.max(jnp.where(bcol == gg, h, -jnp.inf), axis=0)
    max_acc[gg, :] = jnp.maximum(max_acc[gg, :], mg)

  @pl.when(i == nblocks - 1)
  def _finish():
    mean_pool = sum_acc[...] * (1.0 / jnp.maximum(cnt_acc[...], 1.0))
    gft = (jnp.dot(gf_ref[...], wgT_ref[...],
                   preferred_element_type=jnp.float32) + bg_ref[...])
    logits = (jnp.dot(mean_pool, woaT_ref[...],
                      preferred_element_type=jnp.float32)
              + jnp.dot(max_acc[...], wobT_ref[...],
                        preferred_element_type=jnp.float32)
              + jnp.dot(gft, wocT_ref[...],
                        preferred_element_type=jnp.float32)
              + bo_ref[...])
    m = jnp.max(logits, axis=1, keepdims=True)
    lse = m + jnp.log(jnp.sum(jnp.exp(logits - m), axis=1, keepdims=True))
    o_ref[...] = logits - lse


def _make_tc_final(g, n, d, gf, block):
  grid = n // block
  hd = d // 2
  return pl.pallas_call(
      functools.partial(_tc_final_body, g, block),
      grid=(grid,),
      in_specs=[
          pl.BlockSpec((NC, block, hd), lambda i: (0, i, 0)),
          pl.BlockSpec((1, block, LANES), lambda i: (0, i, 0)),
          pl.BlockSpec((NC, block, hd), lambda i: (0, i, 0)),
          pl.BlockSpec((d, d), lambda i: (0, 0)),
          pl.BlockSpec((d, d), lambda i: (0, 0)),
          pl.BlockSpec((1, d), lambda i: (0, 0)),
          pl.BlockSpec((block, 1), lambda i: (i, 0)),
          pl.BlockSpec((g, gf), lambda i: (0, 0)),
          pl.BlockSpec((gf, d), lambda i: (0, 0)),
          pl.BlockSpec((1, d), lambda i: (0, 0)),
          pl.BlockSpec((d, d), lambda i: (0, 0)),
          pl.BlockSpec((d, d), lambda i: (0, 0)),
          pl.BlockSpec((d, d), lambda i: (0, 0)),
          pl.BlockSpec((1, d), lambda i: (0, 0)),
      ],
      out_specs=pl.BlockSpec((g, d), lambda i: (0, 0)),
      out_shape=jax.ShapeDtypeStruct((g, d), jnp.float32),
      scratch_shapes=[
          pltpu.VMEM((g, d), jnp.float32),
          pltpu.VMEM((g, d), jnp.float32),
          pltpu.VMEM((g, d), jnp.float32),
      ],
  )


def kernel(x, edges_idx, batch_idx, g_features, Wl0, bl0, Wr0, Wl1, bl1, Wr1,
           Wg, bg, Wo, bo):
  n, d = x.shape
  e = edges_idx.shape[1]
  g, gf = g_features.shape
  hd = d // 2
  block = 1000

  src = edges_idx[0]
  dst = edges_idx[1]
  batch_col = batch_idx.reshape(n, 1)

  woT = Wo.T
  zpad = jnp.zeros((d, d - 2), jnp.float32)
  woaT = jnp.concatenate([woT[:d], zpad], axis=1)
  wobT = jnp.concatenate([woT[d:2 * d], zpad], axis=1)
  wocT = jnp.concatenate([woT[2 * d:], zpad], axis=1)
  bo_p = jnp.concatenate(
      [bo, jnp.full((d - 2,), -1e30, jnp.float32)]).reshape(1, d)

  sc_agg0 = _make_sc_agg(True, n, e, hd)
  sc_agg1 = _make_sc_agg(False, n, e, hd)
  tc_layer0 = _make_tc_layer(True, n, d, block)
  tc_final = _make_tc_final(g, n, d, gf, block)

  x_halves = _split_halves(x, n, hd)
  agg0, cntp = sc_agg0(x_halves.reshape(2 * n, hd), src, dst)
  h0 = tc_layer0(agg0, cntp, x_halves, Wl0.T, Wr0.T, bl0.reshape(1, d))
  (agg1,) = sc_agg1(h0.reshape(2 * n, hd), src, dst)
  out = tc_final(agg1, cntp, h0, Wl1.T, Wr1.T, bl1.reshape(1, d), batch_col,
                 g_features, Wg.T, bg.reshape(1, d), woaT, wobT, wocT, bo_p)
  return out[:, :2]


# speedup vs baseline: 8.9102x; 1.3245x over previous
import functools

import jax
import jax.numpy as jnp
from jax import lax
from jax.experimental import pallas as pl
from jax.experimental.pallas import tpu as pltpu
from jax.experimental.pallas import tpu_sc as plsc

NC = 2
NS = 16
EDGE_CHUNK = 400
LANES = 16


def _pad_rows(n):
  return ((n + NS * 8 - 1) // (NS * 8)) * NS * 8


def _static_chunks(total, chunk):
  out, off = [], 0
  while off < total:
    sz = min(chunk, total - off)
    out.append((off, sz))
    off += sz
  return out


def _sc_agg_body(with_cnt, n, e, hd, *refs):
  if with_cnt:
    (feat, src2, dst, agg_out, cnt_out,
     sidx_a, sidx_b, didx_a, didx_b, rows_a, rows_b, ones_v,
     acc_sh, cnt_sh, sem_a, sem_b) = refs
  else:
    (feat, src2, dst, agg_out,
     sidx_a, sidx_b, didx_a, didx_b, rows_a, rows_b,
     acc_sh, sem_a, sem_b) = refs
  c = lax.axis_index("c")
  s = lax.axis_index("s")
  n_pad = _pad_rows(n)
  rows_per_tile = n_pad // NS
  ch = EDGE_CHUNK

  def zrow(i, _):
    for j in range(hd // LANES):
      rows_a[i, pl.ds(j * LANES, LANES)] = jnp.zeros((LANES,), jnp.float32)
    return 0
  lax.fori_loop(0, ch, zrow, 0)

  base = s * rows_per_tile
  for off, sz in _static_chunks(rows_per_tile, ch):
    pltpu.sync_copy(rows_a.at[pl.ds(0, sz)], acc_sh.at[pl.ds(base + off, sz)])

  if with_cnt:
    def zone(i, _):
      ones_v[i, :] = jnp.zeros((LANES,), jnp.float32)
      return 0
    lax.fori_loop(0, ch, zone, 0)
    for off, sz in _static_chunks(rows_per_tile, ch):
      pltpu.sync_copy(ones_v.at[pl.ds(0, sz)],
                      cnt_sh.at[pl.ds(base + off, sz)])
    def sone(i, _):
      ones_v[i, :] = jnp.ones((LANES,), jnp.float32)
      return 0
    lax.fori_loop(0, ch, sone, 0)

  plsc.subcore_barrier()

  e_per_tile = e // NS
  n_pairs = e_per_tile // ch // 2
  ebase = s * e_per_tile
  sbase = c * e + ebase

  def wait(rows):
    pltpu.make_async_copy(feat.at[pl.ds(0, ch)], rows,
                          sem_a if rows is rows_a else sem_b).wait()

  pltpu.sync_copy(src2.at[pl.ds(sbase, ch)], sidx_a)
  pltpu.sync_copy(dst.at[pl.ds(ebase, ch)], didx_a)
  pltpu.async_copy(feat.at[sidx_a], rows_a, sem_a)

  def body(p, _):
    off_b = 2 * p * ch + ch
    pltpu.sync_copy(src2.at[pl.ds(sbase + off_b, ch)], sidx_b)
    pltpu.sync_copy(dst.at[pl.ds(ebase + off_b, ch)], didx_b)
    pltpu.async_copy(feat.at[sidx_b], rows_b, sem_b)
    wait(rows_a)
    pltpu.sync_copy(rows_a, acc_sh.at[didx_a], add=True)
    if with_cnt:
      @pl.when(c == 0)
      def _cnt_a():
        pltpu.sync_copy(ones_v, cnt_sh.at[didx_a], add=True)
    @pl.when(p + 1 < n_pairs)
    def _prefetch():
      off_a = 2 * (p + 1) * ch
      pltpu.sync_copy(src2.at[pl.ds(sbase + off_a, ch)], sidx_a)
      pltpu.sync_copy(dst.at[pl.ds(ebase + off_a, ch)], didx_a)
      pltpu.async_copy(feat.at[sidx_a], rows_a, sem_a)
    wait(rows_b)
    pltpu.sync_copy(rows_b, acc_sh.at[didx_b], add=True)
    if with_cnt:
      @pl.when(c == 1)
      def _cnt_b():
        pltpu.sync_copy(ones_v, cnt_sh.at[didx_b], add=True)
    return 0
  lax.fori_loop(0, n_pairs, body, 0)

  plsc.subcore_barrier()

  pltpu.sync_copy(acc_sh.at[pl.ds(base, rows_per_tile)],
                  agg_out.at[c, pl.ds(base, rows_per_tile)])
  if with_cnt:
    pltpu.sync_copy(cnt_sh.at[pl.ds(base, rows_per_tile)],
                    cnt_out.at[c, pl.ds(base, rows_per_tile)])


def _make_sc_agg(with_cnt, n, e, hd):
  mesh = plsc.VectorSubcoreMesh(core_axis_name="c", subcore_axis_name="s")
  n_pad = _pad_rows(n)
  out_type = [jax.ShapeDtypeStruct((NC, n_pad, hd), jnp.float32)]
  scratch = [
      pltpu.VMEM((EDGE_CHUNK,), jnp.int32),
      pltpu.VMEM((EDGE_CHUNK,), jnp.int32),
      pltpu.VMEM((EDGE_CHUNK,), jnp.int32),
      pltpu.VMEM((EDGE_CHUNK,), jnp.int32),
      pltpu.VMEM((EDGE_CHUNK, hd), jnp.float32),
      pltpu.VMEM((EDGE_CHUNK, hd), jnp.float32),
  ]
  if with_cnt:
    out_type.append(jax.ShapeDtypeStruct((NC, n_pad, LANES), jnp.float32))
    scratch.append(pltpu.VMEM((EDGE_CHUNK, LANES), jnp.float32))
  scratch.append(pltpu.VMEM_SHARED((n_pad, hd), jnp.float32))
  if with_cnt:
    scratch.append(pltpu.VMEM_SHARED((n_pad, LANES), jnp.float32))
  scratch.append(pltpu.SemaphoreType.DMA)
  scratch.append(pltpu.SemaphoreType.DMA)
  return pl.kernel(
      functools.partial(_sc_agg_body, with_cnt, n, e, hd),
      out_type=tuple(out_type),
      mesh=mesh,
      scratch_types=tuple(scratch),
      compiler_params=pltpu.CompilerParams(use_tc_tiling_on_sc=False),
  )


def _split_halves(h, n, hd):
  return jnp.stack([h[:, :hd], h[:, hd:]])


def _tc_layer_body(relu, agg_ref, cntp_ref, x_ref, wlT_ref, wrT_ref, bl_ref,
                   o_ref):
  agg = jnp.concatenate([agg_ref[0], agg_ref[1]], axis=1)
  x = jnp.concatenate([x_ref[0], x_ref[1]], axis=1)
  cnt = cntp_ref[0, :, 0] + cntp_ref[1, :, 0]
  rc = 1.0 / jnp.maximum(cnt, 1.0)
  mean = agg * rc[:, None]
  h = (jnp.dot(mean, wlT_ref[...], preferred_element_type=jnp.float32)
       + jnp.dot(x, wrT_ref[...], preferred_element_type=jnp.float32)
       + bl_ref[...])
  h = jnp.maximum(h, 0.0) if relu else h
  hd = h.shape[1] // 2
  o_ref[0] = h[:, :hd]
  o_ref[1] = h[:, hd:]


def _make_tc_layer(relu, n, d, block):
  grid = n // block
  n_pad = _pad_rows(n)
  hd = d // 2
  return pl.pallas_call(
      functools.partial(_tc_layer_body, relu),
      grid=(grid,),
      in_specs=[
          pl.BlockSpec((NC, block, hd), lambda i: (0, i, 0)),
          pl.BlockSpec((NC, block, LANES), lambda i: (0, i, 0)),
          pl.BlockSpec((NC, block, hd), lambda i: (0, i, 0)),
          pl.BlockSpec((d, d), lambda i: (0, 0)),
          pl.BlockSpec((d, d), lambda i: (0, 0)),
          pl.BlockSpec((1, d), lambda i: (0, 0)),
      ],
      out_specs=pl.BlockSpec((NC, block, hd), lambda i: (0, i, 0)),
      out_shape=jax.ShapeDtypeStruct((NC, n, hd), jnp.float32),
  )


def _tc_final_body(g, block, agg_ref, cntp_ref, x_ref, wlT_ref, wrT_ref,
                   bl_ref, batch_ref, gf_ref, wgT_ref, bg_ref, woaT_ref,
                   wobT_ref, wocT_ref, bo_ref, o_ref, sum_acc, cnt_acc,
                   max_acc):
  i = pl.program_id(0)
  nblocks = pl.num_programs(0)

  @pl.when(i == 0)
  def _init():
    sum_acc[...] = jnp.zeros_like(sum_acc)
    cnt_acc[...] = jnp.zeros_like(cnt_acc)
    max_acc[...] = jnp.full_like(max_acc, -jnp.inf)

  agg = jnp.concatenate([agg_ref[0], agg_ref[1]], axis=1)
  x = jnp.concatenate([x_ref[0], x_ref[1]], axis=1)
  cnt = cntp_ref[0, :, 0] + cntp_ref[1, :, 0]
  rc = 1.0 / jnp.maximum(cnt, 1.0)
  mean = agg * rc[:, None]
  h = (jnp.dot(mean, wlT_ref[...], preferred_element_type=jnp.float32)
       + jnp.dot(x, wrT_ref[...], preferred_element_type=jnp.float32)
       + bl_ref[...])

  bcol = batch_ref[...]
  gids = lax.broadcasted_iota(jnp.int32, (block, g), 1)
  onehot = (bcol == gids).astype(jnp.float32)
  sum_acc[...] += lax.dot_general(
      onehot, h, (((0,), (0,)), ((), ())),
      preferred_element_type=jnp.float32)
  cnt_acc[...] += lax.dot_general(
      onehot, jnp.ones_like(h), (((0,), (0,)), ((), ())),
      preferred_element_type=jnp.float32)

  for gg in range(g):
    mg = jnp.max(jnp.where(bcol == gg, h, -jnp.inf), axis=0)
    max_acc[gg, :] = jnp.maximum(max_acc[gg, :], mg)

  @pl.when(i == nblocks - 1)
  def _finish():
    mean_pool = sum_acc[...] * (1.0 / jnp.maximum(cnt_acc[...], 1.0))
    gft = (jnp.dot(gf_ref[...], wgT_ref[...],
                   preferred_element_type=jnp.float32) + bg_ref[...])
    logits = (jnp.dot(mean_pool, woaT_ref[...],
                      preferred_element_type=jnp.float32)
              + jnp.dot(max_acc[...], wobT_ref[...],
                        preferred_element_type=jnp.float32)
              + jnp.dot(gft, wocT_ref[...],
                        preferred_element_type=jnp.float32)
              + bo_ref[...])
    m = jnp.max(logits, axis=1, keepdims=True)
    lse = m + jnp.log(jnp.sum(jnp.exp(logits - m), axis=1, keepdims=True))
    o_ref[...] = logits - lse


def _make_tc_final(g, n, d, gf, block):
  grid = n // block
  hd = d // 2
  return pl.pallas_call(
      functools.partial(_tc_final_body, g, block),
      grid=(grid,),
      in_specs=[
          pl.BlockSpec((NC, block, hd), lambda i: (0, i, 0)),
          pl.BlockSpec((NC, block, LANES), lambda i: (0, i, 0)),
          pl.BlockSpec((NC, block, hd), lambda i: (0, i, 0)),
          pl.BlockSpec((d, d), lambda i: (0, 0)),
          pl.BlockSpec((d, d), lambda i: (0, 0)),
          pl.BlockSpec((1, d), lambda i: (0, 0)),
          pl.BlockSpec((block, 1), lambda i: (i, 0)),
          pl.BlockSpec((g, gf), lambda i: (0, 0)),
          pl.BlockSpec((gf, d), lambda i: (0, 0)),
          pl.BlockSpec((1, d), lambda i: (0, 0)),
          pl.BlockSpec((d, d), lambda i: (0, 0)),
          pl.BlockSpec((d, d), lambda i: (0, 0)),
          pl.BlockSpec((d, d), lambda i: (0, 0)),
          pl.BlockSpec((1, d), lambda i: (0, 0)),
      ],
      out_specs=pl.BlockSpec((g, d), lambda i: (0, 0)),
      out_shape=jax.ShapeDtypeStruct((g, d), jnp.float32),
      scratch_shapes=[
          pltpu.VMEM((g, d), jnp.float32),
          pltpu.VMEM((g, d), jnp.float32),
          pltpu.VMEM((g, d), jnp.float32),
      ],
  )


def kernel(x, edges_idx, batch_idx, g_features, Wl0, bl0, Wr0, Wl1, bl1, Wr1,
           Wg, bg, Wo, bo):
  n, d = x.shape
  e = edges_idx.shape[1]
  g, gf = g_features.shape
  hd = d // 2
  block = 1000

  src = edges_idx[0]
  dst = edges_idx[1]
  batch_col = batch_idx.reshape(n, 1)

  woT = Wo.T
  zpad = jnp.zeros((d, d - 2), jnp.float32)
  woaT = jnp.concatenate([woT[:d], zpad], axis=1)
  wobT = jnp.concatenate([woT[d:2 * d], zpad], axis=1)
  wocT = jnp.concatenate([woT[2 * d:], zpad], axis=1)
  bo_p = jnp.concatenate(
      [bo, jnp.full((d - 2,), -1e30, jnp.float32)]).reshape(1, d)

  sc_agg0 = _make_sc_agg(True, n, e, hd)
  sc_agg1 = _make_sc_agg(False, n, e, hd)
  tc_layer0 = _make_tc_layer(True, n, d, block)
  tc_final = _make_tc_final(g, n, d, gf, block)

  src2 = jnp.concatenate([src, src + n])
  x_halves = _split_halves(x, n, hd)
  agg0, cntp = sc_agg0(x_halves.reshape(2 * n, hd), src2, dst)
  h0 = tc_layer0(agg0, cntp, x_halves, Wl0.T, Wr0.T, bl0.reshape(1, d))
  (agg1,) = sc_agg1(h0.reshape(2 * n, hd), src2, dst)
  out = tc_final(agg1, cntp, h0, Wl1.T, Wr1.T, bl1.reshape(1, d), batch_col,
                 g_features, Wg.T, bg.reshape(1, d), woaT, wobT, wocT, bo_p)
  return out[:, :2]


# speedup vs baseline: 10.6642x; 1.1968x over previous
import functools

import jax
import jax.numpy as jnp
from jax import lax
from jax.experimental import pallas as pl
from jax.experimental.pallas import tpu as pltpu
from jax.experimental.pallas import tpu_sc as plsc

NC = 2
NS = 16
EDGE_CHUNK = 400
LANES = 16


def _pad_rows(n):
  return ((n + NS * 8 - 1) // (NS * 8)) * NS * 8


def _static_chunks(total, chunk):
  out, off = [], 0
  while off < total:
    sz = min(chunk, total - off)
    out.append((off, sz))
    off += sz
  return out


def _sc_agg_body(with_cnt, n, e, hd, *refs):
  if with_cnt:
    (feat, src2, dst, agg_out, cnt_out,
     sidx_a, sidx_b, didx_a, didx_b, rows_a, rows_b, ones_v,
     acc_sh, cnt_sh, sem_a, sem_b) = refs
  else:
    (feat, src2, dst, agg_out,
     sidx_a, sidx_b, didx_a, didx_b, rows_a, rows_b,
     acc_sh, sem_a, sem_b) = refs
  c = lax.axis_index("c")
  s = lax.axis_index("s")
  n_pad = _pad_rows(n)
  rows_per_tile = n_pad // NS
  ch = EDGE_CHUNK

  def zrow(i, _):
    for j in range(hd // LANES):
      rows_a[i, pl.ds(j * LANES, LANES)] = jnp.zeros((LANES,), jnp.float32)
    return 0
  lax.fori_loop(0, ch, zrow, 0)

  base = s * rows_per_tile
  for off, sz in _static_chunks(rows_per_tile, ch):
    pltpu.sync_copy(rows_a.at[pl.ds(0, sz)], acc_sh.at[pl.ds(base + off, sz)])

  if with_cnt:
    def zone(i, _):
      ones_v[i, :] = jnp.zeros((LANES,), jnp.float32)
      return 0
    lax.fori_loop(0, ch, zone, 0)
    for off, sz in _static_chunks(rows_per_tile, ch):
      pltpu.sync_copy(ones_v.at[pl.ds(0, sz)],
                      cnt_sh.at[pl.ds(base + off, sz)])
    def sone(i, _):
      ones_v[i, :] = jnp.ones((LANES,), jnp.float32)
      return 0
    lax.fori_loop(0, ch, sone, 0)

  plsc.subcore_barrier()

  e_per_tile = e // NS
  n_pairs = e_per_tile // ch // 2
  ebase = s * e_per_tile
  sbase = c * e + ebase

  def wait(rows):
    pltpu.make_async_copy(feat.at[pl.ds(0, ch)], rows,
                          sem_a if rows is rows_a else sem_b).wait()

  pltpu.sync_copy(src2.at[pl.ds(sbase, ch)], sidx_a)
  pltpu.sync_copy(dst.at[pl.ds(ebase, ch)], didx_a)
  pltpu.async_copy(feat.at[sidx_a], rows_a, sem_a)

  def body(p, _):
    off_b = 2 * p * ch + ch
    pltpu.sync_copy(src2.at[pl.ds(sbase + off_b, ch)], sidx_b)
    pltpu.sync_copy(dst.at[pl.ds(ebase + off_b, ch)], didx_b)
    pltpu.async_copy(feat.at[sidx_b], rows_b, sem_b)
    wait(rows_a)
    pltpu.sync_copy(rows_a, acc_sh.at[didx_a], add=True)
    if with_cnt:
      @pl.when(c == 0)
      def _cnt_a():
        pltpu.sync_copy(ones_v, cnt_sh.at[didx_a], add=True)
    @pl.when(p + 1 < n_pairs)
    def _prefetch():
      off_a = 2 * (p + 1) * ch
      pltpu.sync_copy(src2.at[pl.ds(sbase + off_a, ch)], sidx_a)
      pltpu.sync_copy(dst.at[pl.ds(ebase + off_a, ch)], didx_a)
      pltpu.async_copy(feat.at[sidx_a], rows_a, sem_a)
    wait(rows_b)
    pltpu.sync_copy(rows_b, acc_sh.at[didx_b], add=True)
    if with_cnt:
      @pl.when(c == 1)
      def _cnt_b():
        pltpu.sync_copy(ones_v, cnt_sh.at[didx_b], add=True)
    return 0
  lax.fori_loop(0, n_pairs, body, 0)

  plsc.subcore_barrier()

  pltpu.sync_copy(acc_sh.at[pl.ds(base, rows_per_tile)],
                  agg_out.at[c, pl.ds(base, rows_per_tile)])
  if with_cnt:
    pltpu.sync_copy(cnt_sh.at[pl.ds(base, rows_per_tile)],
                    cnt_out.at[c, pl.ds(base, rows_per_tile)])


def _make_sc_agg(with_cnt, n, e, hd):
  mesh = plsc.VectorSubcoreMesh(core_axis_name="c", subcore_axis_name="s")
  n_pad = _pad_rows(n)
  out_type = [jax.ShapeDtypeStruct((NC, n_pad, hd), jnp.float32)]
  scratch = [
      pltpu.VMEM((EDGE_CHUNK,), jnp.int32),
      pltpu.VMEM((EDGE_CHUNK,), jnp.int32),
      pltpu.VMEM((EDGE_CHUNK,), jnp.int32),
      pltpu.VMEM((EDGE_CHUNK,), jnp.int32),
      pltpu.VMEM((EDGE_CHUNK, hd), jnp.float32),
      pltpu.VMEM((EDGE_CHUNK, hd), jnp.float32),
  ]
  if with_cnt:
    out_type.append(jax.ShapeDtypeStruct((NC, n_pad, LANES), jnp.float32))
    scratch.append(pltpu.VMEM((EDGE_CHUNK, LANES), jnp.float32))
  scratch.append(pltpu.VMEM_SHARED((n_pad, hd), jnp.float32))
  if with_cnt:
    scratch.append(pltpu.VMEM_SHARED((n_pad, LANES), jnp.float32))
  scratch.append(pltpu.SemaphoreType.DMA)
  scratch.append(pltpu.SemaphoreType.DMA)
  return pl.kernel(
      functools.partial(_sc_agg_body, with_cnt, n, e, hd),
      out_type=tuple(out_type),
      mesh=mesh,
      scratch_types=tuple(scratch),
      compiler_params=pltpu.CompilerParams(use_tc_tiling_on_sc=False),
  )


def _split_halves(h, n, hd):
  return jnp.stack([h[:, :hd], h[:, hd:]])


def _tc_layer_body(relu, agg_ref, cntp_ref, x_ref, wlT_ref, wrT_ref, bl_ref,
                   o_ref):
  agg = jnp.concatenate([agg_ref[0], agg_ref[1]], axis=1)
  x = jnp.concatenate([x_ref[0], x_ref[1]], axis=1)
  cnt = cntp_ref[0, :, 0] + cntp_ref[1, :, 0]
  rc = 1.0 / jnp.maximum(cnt, 1.0)
  mean = agg * rc[:, None]
  h = (jnp.dot(mean, wlT_ref[...], preferred_element_type=jnp.float32)
       + jnp.dot(x, wrT_ref[...], preferred_element_type=jnp.float32)
       + bl_ref[...])
  h = jnp.maximum(h, 0.0) if relu else h
  hd = h.shape[1] // 2
  o_ref[0] = h[:, :hd]
  o_ref[1] = h[:, hd:]


def _make_tc_layer(relu, n, d, block):
  grid = n // block
  n_pad = _pad_rows(n)
  hd = d // 2
  return pl.pallas_call(
      functools.partial(_tc_layer_body, relu),
      grid=(grid,),
      in_specs=[
          pl.BlockSpec((NC, block, hd), lambda i: (0, i, 0)),
          pl.BlockSpec((NC, block, LANES), lambda i: (0, i, 0)),
          pl.BlockSpec((NC, block, hd), lambda i: (0, i, 0)),
          pl.BlockSpec((d, d), lambda i: (0, 0)),
          pl.BlockSpec((d, d), lambda i: (0, 0)),
          pl.BlockSpec((1, d), lambda i: (0, 0)),
      ],
      out_specs=pl.BlockSpec((NC, block, hd), lambda i: (0, i, 0)),
      out_shape=jax.ShapeDtypeStruct((NC, n, hd), jnp.float32),
  )


def _tc_final_body(g, block, agg_ref, cntp_ref, x_ref, wlT_ref, wrT_ref,
                   bl_ref, batch_ref, gf_ref, wgT_ref, bg_ref, woaT_ref,
                   wobT_ref, wocT_ref, bo_ref, o_ref, sum_acc, cnt_acc,
                   max_acc):
  i = pl.program_id(0)
  nblocks = pl.num_programs(0)

  @pl.when(i == 0)
  def _init():
    sum_acc[...] = jnp.zeros_like(sum_acc)
    cnt_acc[...] = jnp.zeros_like(cnt_acc)
    max_acc[...] = jnp.full_like(max_acc, -jnp.inf)

  agg = jnp.concatenate([agg_ref[0], agg_ref[1]], axis=1)
  x = jnp.concatenate([x_ref[0], x_ref[1]], axis=1)
  cnt = cntp_ref[0, :, 0] + cntp_ref[1, :, 0]
  rc = 1.0 / jnp.maximum(cnt, 1.0)
  mean = agg * rc[:, None]
  h = (jnp.dot(mean, wlT_ref[...], preferred_element_type=jnp.float32)
       + jnp.dot(x, wrT_ref[...], preferred_element_type=jnp.float32)
       + bl_ref[...])

  bcol = batch_ref[...]
  gids = lax.broadcasted_iota(jnp.int32, (block, g), 1)
  onehot = (bcol == gids).astype(jnp.float32)
  sum_acc[...] += lax.dot_general(
      onehot, h, (((0,), (0,)), ((), ())),
      preferred_element_type=jnp.float32)
  cnt_acc[...] += lax.dot_general(
      onehot, jnp.ones_like(h), (((0,), (0,)), ((), ())),
      preferred_element_type=jnp.float32)

  d = h.shape[1]
  bb = jnp.broadcast_to(bcol, (block, d))
  hm = h
  k = 1
  while k < block:
    bs = pltpu.roll(bb, k, 0)
    hs = pltpu.roll(hm, k, 0)
    hm = jnp.maximum(hm, jnp.where(bb == bs, hs, -jnp.inf))
    k *= 2
  bnext = pltpu.roll(bb, block - 1, 0)
  rows = lax.broadcasted_iota(jnp.int32, (block, d), 0)
  is_last = (bb != bnext) | (rows == block - 1)
  lastcol = jnp.where(is_last[:, :1], 1.0, 0.0)
  oh_last = onehot * lastcol
  picked = lax.dot_general(oh_last, hm, (((0,), (0,)), ((), ())),
                           preferred_element_type=jnp.float32)
  pres = lax.dot_general(oh_last, jnp.ones_like(h), (((0,), (0,)), ((), ())),
                         preferred_element_type=jnp.float32)
  max_acc[...] = jnp.maximum(
      max_acc[...], jnp.where(pres > 0.5, picked, -jnp.inf))

  @pl.when(i == nblocks - 1)
  def _finish():
    mean_pool = sum_acc[...] * (1.0 / jnp.maximum(cnt_acc[...], 1.0))
    gft = (jnp.dot(gf_ref[...], wgT_ref[...],
                   preferred_element_type=jnp.float32) + bg_ref[...])
    logits = (jnp.dot(mean_pool, woaT_ref[...],
                      preferred_element_type=jnp.float32)
              + jnp.dot(max_acc[...], wobT_ref[...],
                        preferred_element_type=jnp.float32)
              + jnp.dot(gft, wocT_ref[...],
                        preferred_element_type=jnp.float32)
              + bo_ref[...])
    m = jnp.max(logits, axis=1, keepdims=True)
    lse = m + jnp.log(jnp.sum(jnp.exp(logits - m), axis=1, keepdims=True))
    o_ref[...] = logits - lse


def _make_tc_final(g, n, d, gf, block):
  grid = n // block
  hd = d // 2
  return pl.pallas_call(
      functools.partial(_tc_final_body, g, block),
      grid=(grid,),
      in_specs=[
          pl.BlockSpec((NC, block, hd), lambda i: (0, i, 0)),
          pl.BlockSpec((NC, block, LANES), lambda i: (0, i, 0)),
          pl.BlockSpec((NC, block, hd), lambda i: (0, i, 0)),
          pl.BlockSpec((d, d), lambda i: (0, 0)),
          pl.BlockSpec((d, d), lambda i: (0, 0)),
          pl.BlockSpec((1, d), lambda i: (0, 0)),
          pl.BlockSpec((block, 1), lambda i: (i, 0)),
          pl.BlockSpec((g, gf), lambda i: (0, 0)),
          pl.BlockSpec((gf, d), lambda i: (0, 0)),
          pl.BlockSpec((1, d), lambda i: (0, 0)),
          pl.BlockSpec((d, d), lambda i: (0, 0)),
          pl.BlockSpec((d, d), lambda i: (0, 0)),
          pl.BlockSpec((d, d), lambda i: (0, 0)),
          pl.BlockSpec((1, d), lambda i: (0, 0)),
      ],
      out_specs=pl.BlockSpec((g, d), lambda i: (0, 0)),
      out_shape=jax.ShapeDtypeStruct((g, d), jnp.float32),
      scratch_shapes=[
          pltpu.VMEM((g, d), jnp.float32),
          pltpu.VMEM((g, d), jnp.float32),
          pltpu.VMEM((g, d), jnp.float32),
      ],
  )


def kernel(x, edges_idx, batch_idx, g_features, Wl0, bl0, Wr0, Wl1, bl1, Wr1,
           Wg, bg, Wo, bo):
  n, d = x.shape
  e = edges_idx.shape[1]
  g, gf = g_features.shape
  hd = d // 2
  block = 1000

  src = edges_idx[0]
  dst = edges_idx[1]
  batch_col = batch_idx.reshape(n, 1)

  woT = Wo.T
  zpad = jnp.zeros((d, d - 2), jnp.float32)
  woaT = jnp.concatenate([woT[:d], zpad], axis=1)
  wobT = jnp.concatenate([woT[d:2 * d], zpad], axis=1)
  wocT = jnp.concatenate([woT[2 * d:], zpad], axis=1)
  bo_p = jnp.concatenate(
      [bo, jnp.full((d - 2,), -1e30, jnp.float32)]).reshape(1, d)

  sc_agg0 = _make_sc_agg(True, n, e, hd)
  sc_agg1 = _make_sc_agg(False, n, e, hd)
  tc_layer0 = _make_tc_layer(True, n, d, block)
  tc_final = _make_tc_final(g, n, d, gf, block)

  src2 = jnp.concatenate([src, src + n])
  x_halves = _split_halves(x, n, hd)
  agg0, cntp = sc_agg0(x_halves.reshape(2 * n, hd), src2, dst)
  h0 = tc_layer0(agg0, cntp, x_halves, Wl0.T, Wr0.T, bl0.reshape(1, d))
  (agg1,) = sc_agg1(h0.reshape(2 * n, hd), src2, dst)
  out = tc_final(agg1, cntp, h0, Wl1.T, Wr1.T, bl1.reshape(1, d), batch_col,
                 g_features, Wg.T, bg.reshape(1, d), woaT, wobT, wocT, bo_p)
  return out[:, :2]
